# baseline (speedup 1.0000x reference)
import functools

import jax
import jax.numpy as jnp
from jax import lax
from jax.experimental import pallas as pl
from jax.experimental.pallas import tpu as pltpu

_B, _T = 8, 128
_S = 512
_D = 512
_V = 30000
_EXT = _V + 2 * _S
_CW = 512
_NT = 61
_EXTP = _NT * _CW
_H, _DH = 8, 64
_BT = _B * _T
_EPS = 1e-5


def _gram_body(w_ref, b_ref, g_ref, aux_ref):
    i = pl.program_id(0)

    @pl.when(i == 0)
    def _():
        g_ref[...] = jnp.zeros_like(g_ref)
        aux_ref[...] = jnp.zeros_like(aux_ref)

    w = w_ref[...]
    b = b_ref[0]
    g_ref[...] += lax.dot_general(w, w, (((0,), (0,)), ((), ())),
                                  preferred_element_type=jnp.float32)
    aux = aux_ref[...]
    wsum = jnp.sum(w, axis=0, keepdims=True)
    wb = lax.dot_general(b, w, (((1,), (0,)), ((), ())),
                         preferred_element_type=jnp.float32)
    sb = jnp.sum(b)
    sb2 = jnp.sum(b * b)
    aux_ref[0:1, :] = aux[0:1, :] + wsum
    aux_ref[1:2, :] = aux[1:2, :] + wb
    aux_ref[2:3, :] = aux[2:3, :] + jnp.full((1, _D), sb, jnp.float32)
    aux_ref[3:4, :] = aux[3:4, :] + jnp.full((1, _D), sb2, jnp.float32)


def _rowmask_cols(mat):
    a = jnp.abs(mat)
    ones = jnp.ones((8, mat.shape[1]), jnp.float32)
    sums = lax.dot_general(ones, a, (((1,), (1,)), ((), ())),
                           preferred_element_type=jnp.float32)
    return jnp.sign(sums[0:1, :])


def _ln_rows(a):
    m = jnp.mean(a, axis=-1, keepdims=True)
    v = jnp.mean((a - m) ** 2, axis=-1, keepdims=True)
    return (a - m) / jnp.sqrt(v + _EPS)


def _attn_body(x_ref, s1_ref, s2_ref,
               wq1_ref, bq1_ref, wk1_ref, bk1_ref, wv1_ref, bv1_ref,
               wo1_ref, bo1_ref,
               wq2_ref, bq2_ref, wk2_ref, bk2_ref, wv2_ref, bv2_ref,
               wo2_ref, bo2_ref,
               wpt_ref, bp_ref, g_ref, aux_ref, sb_ref, sb2_ref,
               attp_ref, stats_ref):
    x = x_ref[0]
    qmask = jnp.sign(jnp.sum(jnp.abs(x), axis=-1, keepdims=True))

    def one_source(s_ref, wq_ref, bq_ref, wk_ref, bk_ref, wv_ref, bv_ref,
                   wo_ref, bo_ref):
        s = s_ref[0]
        kmask = _rowmask_cols(s)
        q = lax.dot_general(x, wq_ref[...], (((1,), (1,)), ((), ())),
                            preferred_element_type=jnp.float32) + bq_ref[...]
        k = lax.dot_general(s, wk_ref[...], (((1,), (1,)), ((), ())),
                            preferred_element_type=jnp.float32) + bk_ref[...]
        v = lax.dot_general(s, wv_ref[...], (((1,), (1,)), ((), ())),
                            preferred_element_type=jnp.float32) + bv_ref[...]
        raw = lax.dot_general(q, k, (((1,), (1,)), ((), ())),
                              preferred_element_type=jnp.float32)
        raw = raw * kmask * qmask
        att_ln = _ln_rows(raw)
        scale = float(_DH) ** -0.5
        outs = []
        for h in range(_H):
            qh = q[:, h * _DH:(h + 1) * _DH] * scale
            kh = k[:, h * _DH:(h + 1) * _DH]
            vh = v[:, h * _DH:(h + 1) * _DH]
            lg = lax.dot_general(qh, kh, (((1,), (1,)), ((), ())),
                                 preferred_element_type=jnp.float32)
            lg = jnp.where(kmask == 0.0, -jnp.inf, lg)
            mx = jnp.max(lg, axis=-1, keepdims=True)
            ex = jnp.exp(lg - mx)
            sm = ex / jnp.sum(ex, axis=-1, keepdims=True)
            outs.append(lax.dot_general(sm, vh, (((1,), (0,)), ((), ())),
                                        preferred_element_type=jnp.float32))
        o = jnp.concatenate(outs, axis=1)
        c = lax.dot_general(o, wo_ref[...], (((1,), (1,)), ((), ())),
                            preferred_element_type=jnp.float32) + bo_ref[...]
        c = c * qmask
        return att_ln, c

    att1, c1 = one_source(s1_ref, wq1_ref, bq1_ref, wk1_ref, bk1_ref,
                          wv1_ref, bv1_ref, wo1_ref, bo1_ref)
    att2, c2 = one_source(s2_ref, wq2_ref, bq2_ref, wk2_ref, bk2_ref,
                          wv2_ref, bv2_ref, wo2_ref, bo2_ref)

    feat = jnp.concatenate([x, c1, c2], axis=1)
    plog = lax.dot_general(feat, wpt_ref[...], (((1,), (0,)), ((), ())),
                           preferred_element_type=jnp.float32) + bp_ref[...]
    plog = plog[:, 0:3]
    pmx = jnp.max(plog, axis=-1, keepdims=True)
    pex = jnp.exp(plog - pmx)
    p = pex / jnp.sum(pex, axis=-1, keepdims=True)

    g = g_ref[...]
    aux = aux_ref[...]
    wsum = aux[0:1, :]
    wb = aux[1:2, :]
    sb = sb_ref[0, 0]
    sb2 = sb2_ref[0, 0]
    xws = jnp.sum(x * wsum, axis=-1, keepdims=True)
    mean = (xws + sb) / float(_V)
    xg = lax.dot_general(x, g, (((1,), (0,)), ((), ())),
                         preferred_element_type=jnp.float32)
    xgx = jnp.sum(xg * x, axis=-1, keepdims=True)
    xwb = jnp.sum(x * wb, axis=-1, keepdims=True)
    e2 = (xgx + 2.0 * xwb + sb2) / float(_V)
    var = e2 - mean * mean
    rstd = lax.rsqrt(var + _EPS)

    attp_ref[0] = jnp.concatenate([att1 * p[:, 1:2], att2 * p[:, 2:3]],
                                  axis=1)
    zeros = jnp.zeros((_T, 5), jnp.float32)
    stats_ref[0] = jnp.concatenate([mean, rstd, p[:, 0:1], zeros], axis=1)


def _out_body(x_ref, w_ref, b_ref, idxf_ref, attp_ref,
              mean_ref, rstd_ref, p0_ref, out_ref):
    i = pl.program_id(0)
    base = i * _CW
    xw = lax.dot_general(x_ref[...], w_ref[...], (((1,), (1,)), ((), ())),
                         preferred_element_type=jnp.float32)
    xw = xw + b_ref[0]
    coli = lax.broadcasted_iota(jnp.int32, (_BT, _CW), 1) + base
    colmask = jnp.where(coli < _V, 1.0, 0.0)
    dense = (xw - mean_ref[...]) * rstd_ref[...] * p0_ref[...] * colmask

    cols1 = lax.broadcasted_iota(jnp.int32, (2 * _S, _CW), 1) + base
    copies = []
    for b in range(_B):
        idx_b = idxf_ref[b]
        onehot = jnp.where(idx_b == cols1, 1.0, 0.0).astype(jnp.bfloat16)
        ap = attp_ref[b].astype(jnp.bfloat16)
        copies.append(lax.dot_general(ap, onehot, (((1,), (0,)), ((), ())),
                                      preferred_element_type=jnp.float32))
    out_ref[...] = dense + jnp.concatenate(copies, axis=0)


def kernel(tgt_dec_out, src1_key, src1_map_idx, src2_key, src2_map_idx,
           W_out, b_out,
           Wq1, bq1, Wk1, bk1, Wv1, bv1, Wo1, bo1,
           Wq2, bq2, Wk2, bk2, Wv2, bv2, Wo2, bo2,
           Wp, bp):
    f32 = jnp.float32
    x = tgt_dec_out.astype(f32)
    w_pad = jnp.pad(W_out.astype(f32), ((0, _EXTP - _V), (0, 0)))
    b_pad = jnp.pad(b_out.astype(f32), (0, _EXTP - _V)).reshape(_NT, 1, _CW)

    g, aux = pl.pallas_call(
        _gram_body,
        grid=(_NT,),
        in_specs=[
            pl.BlockSpec((_CW, _D), lambda i: (i, 0)),
            pl.BlockSpec((1, 1, _CW), lambda i: (i, 0, 0)),
        ],
        out_specs=[
            pl.BlockSpec((_D, _D), lambda i: (0, 0)),
            pl.BlockSpec((8, _D), lambda i: (0, 0)),
        ],
        out_shape=[
            jax.ShapeDtypeStruct((_D, _D), f32),
            jax.ShapeDtypeStruct((8, _D), f32),
        ],
    )(w_pad, b_pad)

    def vrow(v):
        return v.astype(f32).reshape(1, -1)

    full = lambda s: pl.BlockSpec(s, lambda b: tuple(0 for _ in s))
    attp, stats = pl.pallas_call(
        _attn_body,
        grid=(_B,),
        in_specs=[
            pl.BlockSpec((1, _T, _D), lambda b: (b, 0, 0)),
            pl.BlockSpec((1, _S, _D), lambda b: (b, 0, 0)),
            pl.BlockSpec((1, _S, _D), lambda b: (b, 0, 0)),
            full((_D, _D)), full((1, _D)),
            full((_D, _D)), full((1, _D)),
            full((_D, _D)), full((1, _D)),
            full((_D, _D)), full((1, _D)),
            full((_D, _D)), full((1, _D)),
            full((_D, _D)), full((1, _D)),
            full((_D, _D)), full((1, _D)),
            full((_D, _D)), full((1, _D)),
            full((3 * _D, 8)),
            full((1, 8)),
            full((_D, _D)),
            full((8, _D)),
            pl.BlockSpec(memory_space=pltpu.SMEM),
            pl.BlockSpec(memory_space=pltpu.SMEM),
        ],
        out_specs=[
            pl.BlockSpec((1, _T, 2 * _S), lambda b: (b, 0, 0)),
            pl.BlockSpec((1, _T, 8), lambda b: (b, 0, 0)),
        ],
        out_shape=[
            jax.ShapeDtypeStruct((_B, _T, 2 * _S), f32),
            jax.ShapeDtypeStruct((_B, _T, 8), f32),
        ],
    )(x, src1_key.astype(f32), src2_key.astype(f32),
      Wq1.astype(f32), vrow(bq1), Wk1.astype(f32), vrow(bk1),
      Wv1.astype(f32), vrow(bv1), Wo1.astype(f32), vrow(bo1),
      Wq2.astype(f32), vrow(bq2), Wk2.astype(f32), vrow(bk2),
      Wv2.astype(f32), vrow(bv2), Wo2.astype(f32), vrow(bo2),
      jnp.pad(Wp.astype(f32).T, ((0, 0), (0, 5))),
      jnp.pad(bp.astype(f32).reshape(1, 3), ((0, 0), (0, 5))),
      g, aux, aux[2:3, 0:1], aux[3:4, 0:1])

    stats2 = stats.reshape(_BT, 8)
    mean = stats2[:, 0:1]
    rstd = stats2[:, 1:2]
    p0 = stats2[:, 2:3]
    idxf = jnp.concatenate(
        [src1_map_idx.astype(jnp.int32), src2_map_idx.astype(jnp.int32)],
        axis=1).reshape(_B, 2 * _S, 1)

    out = pl.pallas_call(
        _out_body,
        grid=(_NT,),
        in_specs=[
            pl.BlockSpec((_BT, _D), lambda i: (0, 0)),
            pl.BlockSpec((_CW, _D), lambda i: (i, 0)),
            pl.BlockSpec((1, 1, _CW), lambda i: (i, 0, 0)),
            pl.BlockSpec((_B, 2 * _S, 1), lambda i: (0, 0, 0)),
            pl.BlockSpec((_B, _T, 2 * _S), lambda i: (0, 0, 0)),
            pl.BlockSpec((_BT, 1), lambda i: (0, 0)),
            pl.BlockSpec((_BT, 1), lambda i: (0, 0)),
            pl.BlockSpec((_BT, 1), lambda i: (0, 0)),
        ],
        out_specs=pl.BlockSpec((_BT, _CW), lambda i: (0, i)),
        out_shape=jax.ShapeDtypeStruct((_BT, _EXTP), f32),
    )(x.reshape(_BT, _D), w_pad, b_pad, idxf, attp, mean, rstd, p0)

    return out[:, :_EXT].reshape(_B, _T, _EXT)


# speedup vs baseline: 1.0185x; 1.0185x over previous
import functools

import jax
import jax.numpy as jnp
from jax import lax
from jax.experimental import pallas as pl
from jax.experimental.pallas import tpu as pltpu

_B, _T = 8, 128
_S = 512
_D = 512
_V = 30000
_EXT = _V + 2 * _S
_CW = 512
_NT = 61
_EXTP = _NT * _CW
_H, _DH = 8, 64
_BT = _B * _T
_EPS = 1e-5


def _gram_body(w_ref, b_ref, g_ref, aux_ref):
    i = pl.program_id(0)

    @pl.when(i == 0)
    def _():
        g_ref[...] = jnp.zeros_like(g_ref)
        aux_ref[...] = jnp.zeros_like(aux_ref)

    w = w_ref[...]
    b = b_ref[0]
    g_ref[...] += lax.dot_general(w, w, (((0,), (0,)), ((), ())),
                                  preferred_element_type=jnp.float32)
    aux = aux_ref[...]
    wsum = jnp.sum(w, axis=0, keepdims=True)
    wb = lax.dot_general(b, w, (((1,), (0,)), ((), ())),
                         preferred_element_type=jnp.float32)
    sb = jnp.sum(b)
    sb2 = jnp.sum(b * b)
    aux_ref[0:1, :] = aux[0:1, :] + wsum
    aux_ref[1:2, :] = aux[1:2, :] + wb
    aux_ref[2:3, :] = aux[2:3, :] + jnp.full((1, _D), sb, jnp.float32)
    aux_ref[3:4, :] = aux[3:4, :] + jnp.full((1, _D), sb2, jnp.float32)


def _rowmask_cols(mat):
    a = jnp.abs(mat)
    ones = jnp.ones((8, mat.shape[1]), jnp.float32)
    sums = lax.dot_general(ones, a, (((1,), (1,)), ((), ())),
                           preferred_element_type=jnp.float32)
    return jnp.sign(sums[0:1, :])


def _ln_rows(a):
    m = jnp.mean(a, axis=-1, keepdims=True)
    v = jnp.mean((a - m) ** 2, axis=-1, keepdims=True)
    return (a - m) / jnp.sqrt(v + _EPS)


def _attn_body(x_ref, s1_ref, s2_ref,
               wq1_ref, bq1_ref, wk1_ref, bk1_ref, wv1_ref, bv1_ref,
               wo1_ref, bo1_ref,
               wq2_ref, bq2_ref, wk2_ref, bk2_ref, wv2_ref, bv2_ref,
               wo2_ref, bo2_ref,
               wpt_ref, bp_ref, g_ref, aux_ref, sb_ref, sb2_ref,
               attp_ref, stats_ref):
    x = x_ref[0]
    qmask = jnp.sign(jnp.sum(jnp.abs(x), axis=-1, keepdims=True))

    def one_source(s_ref, wq_ref, bq_ref, wk_ref, bk_ref, wv_ref, bv_ref,
                   wo_ref, bo_ref):
        s = s_ref[0]
        kmask = _rowmask_cols(s)
        q = lax.dot_general(x, wq_ref[...], (((1,), (1,)), ((), ())),
                            preferred_element_type=jnp.float32) + bq_ref[...]
        k = lax.dot_general(s, wk_ref[...], (((1,), (1,)), ((), ())),
                            preferred_element_type=jnp.float32) + bk_ref[...]
        v = lax.dot_general(s, wv_ref[...], (((1,), (1,)), ((), ())),
                            preferred_element_type=jnp.float32) + bv_ref[...]
        raw = lax.dot_general(q, k, (((1,), (1,)), ((), ())),
                              preferred_element_type=jnp.float32)
        raw = raw * kmask * qmask
        att_ln = _ln_rows(raw)
        scale = float(_DH) ** -0.5
        outs = []
        for h in range(_H):
            qh = q[:, h * _DH:(h + 1) * _DH] * scale
            kh = k[:, h * _DH:(h + 1) * _DH]
            vh = v[:, h * _DH:(h + 1) * _DH]
            lg = lax.dot_general(qh, kh, (((1,), (1,)), ((), ())),
                                 preferred_element_type=jnp.float32)
            lg = jnp.where(kmask == 0.0, -jnp.inf, lg)
            mx = jnp.max(lg, axis=-1, keepdims=True)
            ex = jnp.exp(lg - mx)
            sm = ex / jnp.sum(ex, axis=-1, keepdims=True)
            outs.append(lax.dot_general(sm, vh, (((1,), (0,)), ((), ())),
                                        preferred_element_type=jnp.float32))
        o = jnp.concatenate(outs, axis=1)
        c = lax.dot_general(o, wo_ref[...], (((1,), (1,)), ((), ())),
                            preferred_element_type=jnp.float32) + bo_ref[...]
        c = c * qmask
        return att_ln, c

    att1, c1 = one_source(s1_ref, wq1_ref, bq1_ref, wk1_ref, bk1_ref,
                          wv1_ref, bv1_ref, wo1_ref, bo1_ref)
    att2, c2 = one_source(s2_ref, wq2_ref, bq2_ref, wk2_ref, bk2_ref,
                          wv2_ref, bv2_ref, wo2_ref, bo2_ref)

    feat = jnp.concatenate([x, c1, c2], axis=1)
    plog = lax.dot_general(feat, wpt_ref[...], (((1,), (0,)), ((), ())),
                           preferred_element_type=jnp.float32) + bp_ref[...]
    plog = plog[:, 0:3]
    pmx = jnp.max(plog, axis=-1, keepdims=True)
    pex = jnp.exp(plog - pmx)
    p = pex / jnp.sum(pex, axis=-1, keepdims=True)

    g = g_ref[...]
    aux = aux_ref[...]
    wsum = aux[0:1, :]
    wb = aux[1:2, :]
    sb = sb_ref[0, 0]
    sb2 = sb2_ref[0, 0]
    xws = jnp.sum(x * wsum, axis=-1, keepdims=True)
    mean = (xws + sb) / float(_V)
    xg = lax.dot_general(x, g, (((1,), (0,)), ((), ())),
                         preferred_element_type=jnp.float32)
    xgx = jnp.sum(xg * x, axis=-1, keepdims=True)
    xwb = jnp.sum(x * wb, axis=-1, keepdims=True)
    e2 = (xgx + 2.0 * xwb + sb2) / float(_V)
    var = e2 - mean * mean
    rstd = lax.rsqrt(var + _EPS)

    attp_ref[0] = jnp.concatenate([att1 * p[:, 1:2], att2 * p[:, 2:3]],
                                  axis=1)
    zeros = jnp.zeros((_T, 5), jnp.float32)
    stats_ref[0] = jnp.concatenate([mean, rstd, p[:, 0:1], zeros], axis=1)


def _out_body(x_ref, w_ref, b_ref, idxf_ref, attp_ref,
              mean_ref, rstd_ref, p0_ref, out_ref):
    i = pl.program_id(0)
    base = i * _CW
    xw = lax.dot_general(x_ref[...], w_ref[...], (((1,), (1,)), ((), ())),
                         preferred_element_type=jnp.float32)
    xw = xw + b_ref[0]
    coli = lax.broadcasted_iota(jnp.int32, (_BT, _CW), 1) + base
    colmask = jnp.where(coli < _V, 1.0, 0.0)
    dense = (xw - mean_ref[...]) * rstd_ref[...] * p0_ref[...] * colmask

    cols1 = lax.broadcasted_iota(jnp.int32, (2 * _S, _CW), 1) + base
    copies = []
    for b in range(_B):
        idx_b = idxf_ref[b]
        onehot = jnp.where(idx_b == cols1, 1.0, 0.0)
        ap = attp_ref[b]
        copies.append(lax.dot_general(ap, onehot, (((1,), (0,)), ((), ())),
                                      preferred_element_type=jnp.float32))
    out_ref[...] = dense + jnp.concatenate(copies, axis=0)


def kernel(tgt_dec_out, src1_key, src1_map_idx, src2_key, src2_map_idx,
           W_out, b_out,
           Wq1, bq1, Wk1, bk1, Wv1, bv1, Wo1, bo1,
           Wq2, bq2, Wk2, bk2, Wv2, bv2, Wo2, bo2,
           Wp, bp):
    f32 = jnp.float32
    x = tgt_dec_out.astype(f32)
    w_pad = jnp.pad(W_out.astype(f32), ((0, _EXTP - _V), (0, 0)))
    b_pad = jnp.pad(b_out.astype(f32), (0, _EXTP - _V)).reshape(_NT, 1, _CW)

    g, aux = pl.pallas_call(
        _gram_body,
        grid=(_NT,),
        in_specs=[
            pl.BlockSpec((_CW, _D), lambda i: (i, 0)),
            pl.BlockSpec((1, 1, _CW), lambda i: (i, 0, 0)),
        ],
        out_specs=[
            pl.BlockSpec((_D, _D), lambda i: (0, 0)),
            pl.BlockSpec((8, _D), lambda i: (0, 0)),
        ],
        out_shape=[
            jax.ShapeDtypeStruct((_D, _D), f32),
            jax.ShapeDtypeStruct((8, _D), f32),
        ],
    )(w_pad, b_pad)

    def vrow(v):
        return v.astype(f32).reshape(1, -1)

    full = lambda s: pl.BlockSpec(s, lambda b: tuple(0 for _ in s))
    attp, stats = pl.pallas_call(
        _attn_body,
        grid=(_B,),
        in_specs=[
            pl.BlockSpec((1, _T, _D), lambda b: (b, 0, 0)),
            pl.BlockSpec((1, _S, _D), lambda b: (b, 0, 0)),
            pl.BlockSpec((1, _S, _D), lambda b: (b, 0, 0)),
            full((_D, _D)), full((1, _D)),
            full((_D, _D)), full((1, _D)),
            full((_D, _D)), full((1, _D)),
            full((_D, _D)), full((1, _D)),
            full((_D, _D)), full((1, _D)),
            full((_D, _D)), full((1, _D)),
            full((_D, _D)), full((1, _D)),
            full((_D, _D)), full((1, _D)),
            full((3 * _D, 8)),
            full((1, 8)),
            full((_D, _D)),
            full((8, _D)),
            pl.BlockSpec(memory_space=pltpu.SMEM),
            pl.BlockSpec(memory_space=pltpu.SMEM),
        ],
        out_specs=[
            pl.BlockSpec((1, _T, 2 * _S), lambda b: (b, 0, 0)),
            pl.BlockSpec((1, _T, 8), lambda b: (b, 0, 0)),
        ],
        out_shape=[
            jax.ShapeDtypeStruct((_B, _T, 2 * _S), f32),
            jax.ShapeDtypeStruct((_B, _T, 8), f32),
        ],
    )(x, src1_key.astype(f32), src2_key.astype(f32),
      Wq1.astype(f32), vrow(bq1), Wk1.astype(f32), vrow(bk1),
      Wv1.astype(f32), vrow(bv1), Wo1.astype(f32), vrow(bo1),
      Wq2.astype(f32), vrow(bq2), Wk2.astype(f32), vrow(bk2),
      Wv2.astype(f32), vrow(bv2), Wo2.astype(f32), vrow(bo2),
      jnp.pad(Wp.astype(f32).T, ((0, 0), (0, 5))),
      jnp.pad(bp.astype(f32).reshape(1, 3), ((0, 0), (0, 5))),
      g, aux, aux[2:3, 0:1], aux[3:4, 0:1])

    stats2 = stats.reshape(_BT, 8)
    mean = stats2[:, 0:1]
    rstd = stats2[:, 1:2]
    p0 = stats2[:, 2:3]
    idxf = jnp.concatenate(
        [src1_map_idx.astype(jnp.int32), src2_map_idx.astype(jnp.int32)],
        axis=1).reshape(_B, 2 * _S, 1)

    out = pl.pallas_call(
        _out_body,
        grid=(_NT,),
        in_specs=[
            pl.BlockSpec((_BT, _D), lambda i: (0, 0)),
            pl.BlockSpec((_CW, _D), lambda i: (i, 0)),
            pl.BlockSpec((1, 1, _CW), lambda i: (i, 0, 0)),
            pl.BlockSpec((_B, 2 * _S, 1), lambda i: (0, 0, 0)),
            pl.BlockSpec((_B, _T, 2 * _S), lambda i: (0, 0, 0)),
            pl.BlockSpec((_BT, 1), lambda i: (0, 0)),
            pl.BlockSpec((_BT, 1), lambda i: (0, 0)),
            pl.BlockSpec((_BT, 1), lambda i: (0, 0)),
        ],
        out_specs=pl.BlockSpec((_BT, _CW), lambda i: (0, i)),
        out_shape=jax.ShapeDtypeStruct((_BT, _EXTP), f32),
    )(x.reshape(_BT, _D), w_pad, b_pad, idxf, attp, mean, rstd, p0)

    return out[:, :_EXT].reshape(_B, _T, _EXT)


# speedup vs baseline: 1.3982x; 1.3727x over previous
import functools

import jax
import jax.numpy as jnp
from jax import lax
from jax.experimental import pallas as pl
from jax.experimental.pallas import tpu as pltpu

_B, _T = 8, 128
_S = 512
_D = 512
_V = 30000
_EXT = _V + 2 * _S
_CW = 512
_NT = 61
_EXTP = _NT * _CW
_H, _DH = 8, 64
_BT = _B * _T
_EPS = 1e-5


def _gram_body(w_ref, b_ref, g_ref, aux_ref):
    i = pl.program_id(0)

    @pl.when(i == 0)
    def _():
        g_ref[...] = jnp.zeros_like(g_ref)
        aux_ref[...] = jnp.zeros_like(aux_ref)

    w = w_ref[...]
    b = b_ref[0]
    g_ref[...] += lax.dot_general(w, w, (((0,), (0,)), ((), ())),
                                  preferred_element_type=jnp.float32)
    aux = aux_ref[...]
    wsum = jnp.sum(w, axis=0, keepdims=True)
    wb = lax.dot_general(b, w, (((1,), (0,)), ((), ())),
                         preferred_element_type=jnp.float32)
    sb = jnp.sum(b)
    sb2 = jnp.sum(b * b)
    aux_ref[0:1, :] = aux[0:1, :] + wsum
    aux_ref[1:2, :] = aux[1:2, :] + wb
    aux_ref[2:3, :] = aux[2:3, :] + jnp.full((1, _D), sb, jnp.float32)
    aux_ref[3:4, :] = aux[3:4, :] + jnp.full((1, _D), sb2, jnp.float32)


def _rowmask_cols(mat):
    a = jnp.abs(mat)
    ones = jnp.ones((8, mat.shape[1]), jnp.float32)
    sums = lax.dot_general(ones, a, (((1,), (1,)), ((), ())),
                           preferred_element_type=jnp.float32)
    return jnp.sign(sums[0:1, :])


def _ln_rows(a):
    m = jnp.mean(a, axis=-1, keepdims=True)
    v = jnp.mean((a - m) ** 2, axis=-1, keepdims=True)
    return (a - m) / jnp.sqrt(v + _EPS)


def _attn_body(x_ref, s1_ref, s2_ref,
               wq1_ref, bq1_ref, wk1_ref, bk1_ref, wv1_ref, bv1_ref,
               wo1_ref, bo1_ref,
               wq2_ref, bq2_ref, wk2_ref, bk2_ref, wv2_ref, bv2_ref,
               wo2_ref, bo2_ref,
               wpt_ref, bp_ref, g_ref, aux_ref, sb_ref, sb2_ref,
               attp_ref, stats_ref):
    x = x_ref[0]
    qmask = jnp.sign(jnp.sum(jnp.abs(x), axis=-1, keepdims=True))

    def one_source(s_ref, wq_ref, bq_ref, wk_ref, bk_ref, wv_ref, bv_ref,
                   wo_ref, bo_ref):
        s = s_ref[0]
        kmask = _rowmask_cols(s)
        q = lax.dot_general(x, wq_ref[...], (((1,), (1,)), ((), ())),
                            preferred_element_type=jnp.float32) + bq_ref[...]
        k = lax.dot_general(s, wk_ref[...], (((1,), (1,)), ((), ())),
                            preferred_element_type=jnp.float32) + bk_ref[...]
        v = lax.dot_general(s, wv_ref[...], (((1,), (1,)), ((), ())),
                            preferred_element_type=jnp.float32) + bv_ref[...]
        raw = lax.dot_general(q, k, (((1,), (1,)), ((), ())),
                              preferred_element_type=jnp.float32)
        raw = raw * kmask * qmask
        att_ln = _ln_rows(raw)
        scale = float(_DH) ** -0.5
        outs = []
        for h in range(_H):
            qh = q[:, h * _DH:(h + 1) * _DH] * scale
            kh = k[:, h * _DH:(h + 1) * _DH]
            vh = v[:, h * _DH:(h + 1) * _DH]
            lg = lax.dot_general(qh, kh, (((1,), (1,)), ((), ())),
                                 preferred_element_type=jnp.float32)
            lg = jnp.where(kmask == 0.0, -jnp.inf, lg)
            mx = jnp.max(lg, axis=-1, keepdims=True)
            ex = jnp.exp(lg - mx)
            sm = ex / jnp.sum(ex, axis=-1, keepdims=True)
            outs.append(lax.dot_general(sm, vh, (((1,), (0,)), ((), ())),
                                        preferred_element_type=jnp.float32))
        o = jnp.concatenate(outs, axis=1)
        c = lax.dot_general(o, wo_ref[...], (((1,), (1,)), ((), ())),
                            preferred_element_type=jnp.float32) + bo_ref[...]
        c = c * qmask
        return att_ln, c

    att1, c1 = one_source(s1_ref, wq1_ref, bq1_ref, wk1_ref, bk1_ref,
                          wv1_ref, bv1_ref, wo1_ref, bo1_ref)
    att2, c2 = one_source(s2_ref, wq2_ref, bq2_ref, wk2_ref, bk2_ref,
                          wv2_ref, bv2_ref, wo2_ref, bo2_ref)

    feat = jnp.concatenate([x, c1, c2], axis=1)
    plog = lax.dot_general(feat, wpt_ref[...], (((1,), (0,)), ((), ())),
                           preferred_element_type=jnp.float32) + bp_ref[...]
    plog = plog[:, 0:3]
    pmx = jnp.max(plog, axis=-1, keepdims=True)
    pex = jnp.exp(plog - pmx)
    p = pex / jnp.sum(pex, axis=-1, keepdims=True)

    g = g_ref[...]
    aux = aux_ref[...]
    wsum = aux[0:1, :]
    wb = aux[1:2, :]
    sb = sb_ref[0, 0]
    sb2 = sb2_ref[0, 0]
    xws = jnp.sum(x * wsum, axis=-1, keepdims=True)
    mean = (xws + sb) / float(_V)
    xg = lax.dot_general(x, g, (((1,), (0,)), ((), ())),
                         preferred_element_type=jnp.float32)
    xgx = jnp.sum(xg * x, axis=-1, keepdims=True)
    xwb = jnp.sum(x * wb, axis=-1, keepdims=True)
    e2 = (xgx + 2.0 * xwb + sb2) / float(_V)
    var = e2 - mean * mean
    rstd = lax.rsqrt(var + _EPS)

    attp_ref[0] = jnp.concatenate([att1 * p[:, 1:2], att2 * p[:, 2:3]],
                                  axis=1)
    zeros = jnp.zeros((_T, 5), jnp.float32)
    stats_ref[0] = jnp.concatenate([mean, rstd, p[:, 0:1], zeros], axis=1)


def _out_body(x_ref, w_ref, b_ref, idxf_ref, attp_ref,
              mean_ref, rstd_ref, p0_ref, out_ref):
    i = pl.program_id(0)
    base = i * _CW
    xw = lax.dot_general(x_ref[...], w_ref[...], (((1,), (1,)), ((), ())),
                         preferred_element_type=jnp.float32)
    xw = xw + b_ref[0]
    coli = lax.broadcasted_iota(jnp.int32, (_BT, _CW), 1) + base
    dense = jnp.where(coli < _V,
                      (xw - mean_ref[...]) * rstd_ref[...] * p0_ref[...],
                      0.0)

    cols1 = lax.broadcasted_iota(jnp.int32, (2 * _S, _CW), 1) + base
    copies = []
    for b in range(_B):
        idx_b = idxf_ref[b]
        onehot = jnp.where(idx_b == cols1, 1.0, 0.0)
        ap = attp_ref[b]
        copies.append(lax.dot_general(ap, onehot, (((1,), (0,)), ((), ())),
                                      preferred_element_type=jnp.float32))
    out_ref[...] = dense + jnp.concatenate(copies, axis=0)


def kernel(tgt_dec_out, src1_key, src1_map_idx, src2_key, src2_map_idx,
           W_out, b_out,
           Wq1, bq1, Wk1, bk1, Wv1, bv1, Wo1, bo1,
           Wq2, bq2, Wk2, bk2, Wv2, bv2, Wo2, bo2,
           Wp, bp):
    f32 = jnp.float32
    x = tgt_dec_out.astype(f32)
    w = W_out.astype(f32)
    b_vec = b_out.astype(f32)
    b_pad = jnp.pad(b_vec, (0, _EXTP - _V)).reshape(_NT, 1, _CW)

    kt = 2000
    g, aux = pl.pallas_call(
        _gram_body,
        grid=(_V // kt,),
        in_specs=[
            pl.BlockSpec((kt, _D), lambda i: (i, 0)),
            pl.BlockSpec((1, 1, kt), lambda i: (i, 0, 0)),
        ],
        out_specs=[
            pl.BlockSpec((_D, _D), lambda i: (0, 0)),
            pl.BlockSpec((8, _D), lambda i: (0, 0)),
        ],
        out_shape=[
            jax.ShapeDtypeStruct((_D, _D), f32),
            jax.ShapeDtypeStruct((8, _D), f32),
        ],
    )(w, b_vec.reshape(_V // kt, 1, kt))

    def vrow(v):
        return v.astype(f32).reshape(1, -1)

    full = lambda s: pl.BlockSpec(s, lambda b: tuple(0 for _ in s))
    attp, stats = pl.pallas_call(
        _attn_body,
        grid=(_B,),
        in_specs=[
            pl.BlockSpec((1, _T, _D), lambda b: (b, 0, 0)),
            pl.BlockSpec((1, _S, _D), lambda b: (b, 0, 0)),
            pl.BlockSpec((1, _S, _D), lambda b: (b, 0, 0)),
            full((_D, _D)), full((1, _D)),
            full((_D, _D)), full((1, _D)),
            full((_D, _D)), full((1, _D)),
            full((_D, _D)), full((1, _D)),
            full((_D, _D)), full((1, _D)),
            full((_D, _D)), full((1, _D)),
            full((_D, _D)), full((1, _D)),
            full((_D, _D)), full((1, _D)),
            full((3 * _D, 8)),
            full((1, 8)),
            full((_D, _D)),
            full((8, _D)),
            pl.BlockSpec(memory_space=pltpu.SMEM),
            pl.BlockSpec(memory_space=pltpu.SMEM),
        ],
        out_specs=[
            pl.BlockSpec((1, _T, 2 * _S), lambda b: (b, 0, 0)),
            pl.BlockSpec((1, _T, 8), lambda b: (b, 0, 0)),
        ],
        out_shape=[
            jax.ShapeDtypeStruct((_B, _T, 2 * _S), f32),
            jax.ShapeDtypeStruct((_B, _T, 8), f32),
        ],
    )(x, src1_key.astype(f32), src2_key.astype(f32),
      Wq1.astype(f32), vrow(bq1), Wk1.astype(f32), vrow(bk1),
      Wv1.astype(f32), vrow(bv1), Wo1.astype(f32), vrow(bo1),
      Wq2.astype(f32), vrow(bq2), Wk2.astype(f32), vrow(bk2),
      Wv2.astype(f32), vrow(bv2), Wo2.astype(f32), vrow(bo2),
      jnp.pad(Wp.astype(f32).T, ((0, 0), (0, 5))),
      jnp.pad(bp.astype(f32).reshape(1, 3), ((0, 0), (0, 5))),
      g, aux, aux[2:3, 0:1], aux[3:4, 0:1])

    stats2 = stats.reshape(_BT, 8)
    mean = stats2[:, 0:1]
    rstd = stats2[:, 1:2]
    p0 = stats2[:, 2:3]
    idxf = jnp.concatenate(
        [src1_map_idx.astype(jnp.int32), src2_map_idx.astype(jnp.int32)],
        axis=1).reshape(_B, 2 * _S, 1)

    nwt = _V // _CW
    out = pl.pallas_call(
        _out_body,
        grid=(_NT,),
        in_specs=[
            pl.BlockSpec((_BT, _D), lambda i: (0, 0)),
            pl.BlockSpec((_CW, _D), lambda i: (jnp.minimum(i, nwt), 0)),
            pl.BlockSpec((1, 1, _CW), lambda i: (i, 0, 0)),
            pl.BlockSpec((_B, 2 * _S, 1), lambda i: (0, 0, 0)),
            pl.BlockSpec((_B, _T, 2 * _S), lambda i: (0, 0, 0)),
            pl.BlockSpec((_BT, 1), lambda i: (0, 0)),
            pl.BlockSpec((_BT, 1), lambda i: (0, 0)),
            pl.BlockSpec((_BT, 1), lambda i: (0, 0)),
        ],
        out_specs=pl.BlockSpec((_BT, _CW), lambda i: (0, i)),
        out_shape=jax.ShapeDtypeStruct((_BT, _EXT), f32),
    )(x.reshape(_BT, _D), w, b_pad, idxf, attp, mean, rstd, p0)

    return out.reshape(_B, _T, _EXT)
